# baseline (speedup 1.0000x reference)
import functools

import jax
import jax.numpy as jnp
from jax import lax
from jax.experimental import pallas as pl
from jax.experimental.pallas import tpu as pltpu
from jax.experimental.pallas import tpu_sc as plsc

_NUM_WORKERS = 32


def _pool_sc(inp, lengths, table):
    maxlen, batch = inp.shape
    _, dim = table.shape
    bpw = batch // _NUM_WORKERS

    mesh = plsc.VectorSubcoreMesh(core_axis_name="c", subcore_axis_name="s")

    @functools.partial(
        pl.kernel,
        mesh=mesh,
        out_type=jax.ShapeDtypeStruct((batch, dim), jnp.float32),
        compiler_params=pltpu.CompilerParams(use_tc_tiling_on_sc=False),
        scratch_types=[
            pltpu.VMEM((maxlen, bpw), jnp.int32),
            pltpu.VMEM((bpw,), jnp.int32),
            pltpu.VMEM((bpw, dim), jnp.float32),
            pltpu.VMEM((bpw, dim), jnp.float32),
            pltpu.VMEM((bpw, dim), jnp.float32),
            pltpu.SemaphoreType.DMA,
            pltpu.SemaphoreType.DMA,
        ],
    )
    def k(inp_hbm, len_hbm, table_hbm, out_hbm,
          idx_v, lens_v, acc_v, rows0, rows1, sem0, sem1):
        wid = lax.axis_index("c") * 16 + lax.axis_index("s")
        base = wid * bpw

        pltpu.sync_copy(inp_hbm.at[:, pl.ds(base, bpw)], idx_v)
        pltpu.sync_copy(len_hbm.at[pl.ds(base, bpw)], lens_v)

        zeros_f = jnp.zeros((16,), jnp.float32)
        zeros_i = jnp.zeros((16,), jnp.int32)

        @pl.loop(0, bpw // 16)
        def _(j):
            lv = lens_v[pl.ds(j * 16, 16)]

            @pl.loop(0, maxlen)
            def _(t):
                iv = idx_v[t, pl.ds(j * 16, 16)]
                idx_v[t, pl.ds(j * 16, 16)] = jnp.where(lv > t, iv, zeros_i)

        @pl.loop(0, bpw)
        def _(i):
            @pl.loop(0, dim // 16)
            def _(j):
                acc_v[i, pl.ds(j * 16, 16)] = zeros_f

        def start(t, buf, sem):
            pltpu.make_async_copy(table_hbm.at[idx_v.at[t]], buf, sem).start()

        def finish(t, buf, sem):
            pltpu.make_async_copy(table_hbm.at[idx_v.at[t]], buf, sem).wait()

        def accum(buf):
            @pl.loop(0, bpw)
            def _(i):
                for j in range(dim // 16):
                    sl = pl.ds(j * 16, 16)
                    plsc.addupdate(acc_v.at[i, sl], buf[i, sl])

        start(0, rows0, sem0)

        @pl.loop(0, maxlen, step=2)
        def _(t):
            start(t + 1, rows1, sem1)
            finish(t, rows0, sem0)
            accum(rows0)

            @pl.when(t + 2 < maxlen)
            def _():
                start(t + 2, rows0, sem0)

            finish(t + 1, rows1, sem1)
            accum(rows1)

        pltpu.sync_copy(acc_v, out_hbm.at[pl.ds(base, bpw)])

    return k(inp, lengths, table)


def _ffnn_body(maxlen, sums_ref, len_ref, row0_ref, w1_ref, b1_ref,
               w2_ref, b2_ref, out_ref):
    lf = len_ref[...].astype(jnp.float32)
    sums = sums_ref[...] - (maxlen - lf) * row0_ref[...]
    vec = sums / jnp.maximum(lf, 1.0)
    h = jnp.dot(vec, w1_ref[...], preferred_element_type=jnp.float32)
    h = jnp.maximum(h + b1_ref[...], 0.0)
    logits = jnp.dot(h, w2_ref[...], preferred_element_type=jnp.float32)
    logits = logits + b2_ref[...]
    m = jnp.max(logits, axis=1, keepdims=True)
    lse = jnp.log(jnp.sum(jnp.exp(logits - m), axis=1, keepdims=True)) + m
    out_ref[...] = logits - lse


def kernel(inp, lengths, table, W1, b1, W2, b2):
    maxlen, batch = inp.shape
    out_dim = W2.shape[1]

    sums = _pool_sc(inp.astype(jnp.int32), lengths.astype(jnp.int32), table)

    row0 = lax.slice(table, (0, 0), (1, table.shape[1]))
    return pl.pallas_call(
        functools.partial(_ffnn_body, float(maxlen)),
        out_shape=jax.ShapeDtypeStruct((batch, out_dim), jnp.float32),
    )(sums, lengths.reshape(batch, 1), row0,
      W1, b1.reshape(1, -1), W2, b2.reshape(1, -1))


# speedup vs baseline: 1.0005x; 1.0005x over previous
import functools

import jax
import jax.numpy as jnp
from jax import lax
from jax.experimental import pallas as pl
from jax.experimental.pallas import tpu as pltpu
from jax.experimental.pallas import tpu_sc as plsc

_NUM_WORKERS = 32
_NBUF = 8


def _pool_sc(inp, lengths, table):
    maxlen, batch = inp.shape
    _, dim = table.shape
    bpw = batch // _NUM_WORKERS

    mesh = plsc.VectorSubcoreMesh(core_axis_name="c", subcore_axis_name="s")

    @functools.partial(
        pl.kernel,
        mesh=mesh,
        out_type=jax.ShapeDtypeStruct((batch, dim), jnp.float32),
        compiler_params=pltpu.CompilerParams(use_tc_tiling_on_sc=False),
        scratch_types=(
            [
                pltpu.VMEM((maxlen, bpw), jnp.int32),
                pltpu.VMEM((bpw,), jnp.int32),
                pltpu.VMEM((bpw, dim), jnp.float32),
            ]
            + [pltpu.VMEM((bpw, dim), jnp.float32) for _ in range(_NBUF)]
            + [pltpu.SemaphoreType.DMA for _ in range(_NBUF)]
        ),
    )
    def k(inp_hbm, len_hbm, table_hbm, out_hbm,
          idx_v, lens_v, acc_v, *rest):
        rows = rest[:_NBUF]
        sems = rest[_NBUF:]
        wid = lax.axis_index("c") * 16 + lax.axis_index("s")
        base = wid * bpw

        pltpu.sync_copy(inp_hbm.at[:, pl.ds(base, bpw)], idx_v)
        pltpu.sync_copy(len_hbm.at[pl.ds(base, bpw)], lens_v)

        zeros_f = jnp.zeros((16,), jnp.float32)
        zeros_i = jnp.zeros((16,), jnp.int32)

        @pl.loop(0, bpw // 16)
        def _(j):
            lv = lens_v[pl.ds(j * 16, 16)]

            @pl.loop(0, maxlen, unroll=4)
            def _(t):
                iv = idx_v[t, pl.ds(j * 16, 16)]
                idx_v[t, pl.ds(j * 16, 16)] = jnp.where(lv > t, iv, zeros_i)

        @pl.loop(0, bpw, unroll=4)
        def _(i):
            @pl.loop(0, dim // 16)
            def _(j):
                acc_v[i, pl.ds(j * 16, 16)] = zeros_f

        def start(t, buf, sem):
            pltpu.make_async_copy(table_hbm.at[idx_v.at[t]], buf, sem).start()

        def finish(t, buf, sem):
            pltpu.make_async_copy(table_hbm.at[idx_v.at[t]], buf, sem).wait()

        def accum(buf):
            @pl.loop(0, bpw, unroll=4)
            def _(i):
                for j in range(dim // 16):
                    sl = pl.ds(j * 16, 16)
                    plsc.addupdate(acc_v.at[i, sl], buf[i, sl])

        for b in range(_NBUF):
            start(b, rows[b], sems[b])

        @pl.loop(0, maxlen, step=_NBUF)
        def _(t):
            for b in range(_NBUF):
                finish(t + b, rows[b], sems[b])
                accum(rows[b])

                @pl.when(t + b + _NBUF < maxlen)
                def _(b=b):
                    start(t + b + _NBUF, rows[b], sems[b])

        pltpu.sync_copy(acc_v, out_hbm.at[pl.ds(base, bpw)])

    return k(inp, lengths, table)


def _ffnn_body(maxlen, sums_ref, len_ref, row0_ref, w1_ref, b1_ref,
               w2_ref, b2_ref, out_ref):
    lf = len_ref[...].astype(jnp.float32)
    sums = sums_ref[...] - (maxlen - lf) * row0_ref[...]
    vec = sums / jnp.maximum(lf, 1.0)
    h = jnp.dot(vec, w1_ref[...], preferred_element_type=jnp.float32)
    h = jnp.maximum(h + b1_ref[...], 0.0)
    logits = jnp.dot(h, w2_ref[...], preferred_element_type=jnp.float32)
    logits = logits + b2_ref[...]
    m = jnp.max(logits, axis=1, keepdims=True)
    lse = jnp.log(jnp.sum(jnp.exp(logits - m), axis=1, keepdims=True)) + m
    out_ref[...] = logits - lse


def kernel(inp, lengths, table, W1, b1, W2, b2):
    maxlen, batch = inp.shape
    out_dim = W2.shape[1]

    sums = _pool_sc(inp.astype(jnp.int32), lengths.astype(jnp.int32), table)

    row0 = lax.slice(table, (0, 0), (1, table.shape[1]))
    return pl.pallas_call(
        functools.partial(_ffnn_body, float(maxlen)),
        out_shape=jax.ShapeDtypeStruct((batch, out_dim), jnp.float32),
    )(sums, lengths.reshape(batch, 1), row0,
      W1, b1.reshape(1, -1), W2, b2.reshape(1, -1))


# speedup vs baseline: 7.5697x; 7.5660x over previous
import functools

import jax
import jax.numpy as jnp
from jax import lax
from jax.experimental import pallas as pl
from jax.experimental.pallas import tpu as pltpu
from jax.experimental.pallas import tpu_sc as plsc

_NUM_WORKERS = 32
_NBUF = 8


def _pool_sc(inp, lengths, table):
    maxlen, batch = inp.shape
    _, dim = table.shape
    bpw = batch // _NUM_WORKERS

    mesh = plsc.VectorSubcoreMesh(core_axis_name="c", subcore_axis_name="s")

    @functools.partial(
        pl.kernel,
        mesh=mesh,
        out_type=(
            jax.ShapeDtypeStruct((batch, dim), jnp.float32),
            jax.ShapeDtypeStruct((batch, dim), jnp.float32),
        ),
        compiler_params=pltpu.CompilerParams(use_tc_tiling_on_sc=False),
        scratch_types=(
            [
                pltpu.VMEM((maxlen, bpw), jnp.int32),
                pltpu.VMEM((bpw,), jnp.int32),
                pltpu.VMEM((bpw, dim), jnp.float32),
            ]
            + [pltpu.VMEM((bpw, dim), jnp.float32) for _ in range(_NBUF)]
            + [pltpu.SemaphoreType.DMA for _ in range(_NBUF)]
        ),
    )
    def k(inp_hbm, len_hbm, table_hbm, out_hbm, dummy_hbm,
          idx_v, lens_v, acc_v, *rest):
        rows = rest[:_NBUF]
        sems = rest[_NBUF:]
        wid = lax.axis_index("c") * 16 + lax.axis_index("s")
        base = wid * bpw

        pltpu.sync_copy(inp_hbm.at[:, pl.ds(base, bpw)], idx_v)
        pltpu.sync_copy(len_hbm.at[pl.ds(base, bpw)], lens_v)

        zeros_f = jnp.zeros((16,), jnp.float32)

        @pl.loop(0, bpw // 16)
        def _(j):
            lv = lens_v[pl.ds(j * 16, 16)]
            dv = idx_v[0, pl.ds(j * 16, 16)]

            @pl.loop(1, maxlen, unroll=4)
            def _(t):
                iv = idx_v[t, pl.ds(j * 16, 16)]
                idx_v[t, pl.ds(j * 16, 16)] = jnp.where(lv > t, iv, dv)

        @pl.loop(0, bpw, unroll=4)
        def _(i):
            @pl.loop(0, dim // 16)
            def _(j):
                acc_v[i, pl.ds(j * 16, 16)] = zeros_f

        def start(t, buf, sem):
            pltpu.make_async_copy(table_hbm.at[idx_v.at[t]], buf, sem).start()

        def finish(t, buf, sem):
            pltpu.make_async_copy(table_hbm.at[idx_v.at[t]], buf, sem).wait()

        def accum(buf):
            @pl.loop(0, bpw, unroll=4)
            def _(i):
                for j in range(dim // 16):
                    sl = pl.ds(j * 16, 16)
                    plsc.addupdate(acc_v.at[i, sl], buf[i, sl])

        for b in range(_NBUF):
            start(b, rows[b], sems[b])

        @pl.loop(0, maxlen, step=_NBUF)
        def _(t):
            for b in range(_NBUF):
                finish(t + b, rows[b], sems[b])
                accum(rows[b])

                @pl.when(t + b == 0)
                def _(b=b):
                    pltpu.sync_copy(rows[b], dummy_hbm.at[pl.ds(base, bpw)])

                @pl.when(t + b + _NBUF < maxlen)
                def _(b=b):
                    start(t + b + _NBUF, rows[b], sems[b])

        pltpu.sync_copy(acc_v, out_hbm.at[pl.ds(base, bpw)])

    return k(inp, lengths, table)


def _ffnn_body(maxlen, sums_ref, len_ref, dummy_ref, w1_ref, b1_ref,
               w2_ref, b2_ref, out_ref):
    lf = len_ref[...].astype(jnp.float32)
    sums = sums_ref[...] - (maxlen - lf) * dummy_ref[...]
    vec = sums / jnp.maximum(lf, 1.0)
    h = jnp.dot(vec, w1_ref[...], preferred_element_type=jnp.float32)
    h = jnp.maximum(h + b1_ref[...], 0.0)
    logits = jnp.dot(h, w2_ref[...], preferred_element_type=jnp.float32)
    logits = logits + b2_ref[...]
    m = jnp.max(logits, axis=1, keepdims=True)
    lse = jnp.log(jnp.sum(jnp.exp(logits - m), axis=1, keepdims=True)) + m
    out_ref[...] = logits - lse


def kernel(inp, lengths, table, W1, b1, W2, b2):
    maxlen, batch = inp.shape
    out_dim = W2.shape[1]

    sums, dummy = _pool_sc(inp.astype(jnp.int32), lengths.astype(jnp.int32),
                           table)

    return pl.pallas_call(
        functools.partial(_ffnn_body, float(maxlen)),
        out_shape=jax.ShapeDtypeStruct((batch, out_dim), jnp.float32),
    )(sums, lengths.reshape(batch, 1), dummy,
      W1, b1.reshape(1, -1), W2, b2.reshape(1, -1))
